# baseline (speedup 1.0000x reference)
import functools

import jax
import jax.numpy as jnp
from jax import lax
from jax.experimental import pallas as pl
from jax.experimental.pallas import tpu as pltpu
from jax.experimental.pallas import tpu_sc as plsc

NC = 2
NS = 16
NW = NC * NS
CHUNK = 96


def _positional_encoding(seq_len, d_model):
    pos = jnp.arange(seq_len, dtype=jnp.float32)[:, None]
    i = jnp.arange(d_model // 2, dtype=jnp.float32)[None, :]
    angle = pos / jnp.power(10000.0, (2.0 * i) / d_model)
    pe = jnp.zeros((seq_len, d_model), dtype=jnp.float32)
    pe = pe.at[:, 0::2].set(jnp.sin(angle))
    pe = pe.at[:, 1::2].set(jnp.cos(angle))
    return pe


@functools.partial(jax.jit, static_argnames=("n_rows", "d"))
def _sc_gather(table, idx3, *, n_rows, d):
    rpw = n_rows // NW
    n_chunks = rpw // CHUNK
    mesh = plsc.VectorSubcoreMesh(core_axis_name="c", subcore_axis_name="s")

    @functools.partial(
        pl.kernel,
        out_type=jax.ShapeDtypeStruct((n_rows, d), jnp.float32),
        mesh=mesh,
        scratch_types=[
            pltpu.VMEM((n_chunks, CHUNK), jnp.int32),
            pltpu.VMEM((2, CHUNK, d), jnp.float32),
            pltpu.SemaphoreType.DMA,
            pltpu.SemaphoreType.DMA,
        ],
    )
    def k(table_hbm, idx_hbm, out_hbm, idx_v, buf_v, gsem, ssem):
        wid = lax.axis_index("s") * NC + lax.axis_index("c")
        base = wid * rpw
        pltpu.sync_copy(idx_hbm.at[wid], idx_v)

        def gather(c, slot):
            return pltpu.async_copy(table_hbm.at[idx_v.at[c]],
                                    buf_v.at[slot], gsem)

        def writeback(c, slot):
            return pltpu.async_copy(buf_v.at[slot],
                                    out_hbm.at[pl.ds(base + c * CHUNK, CHUNK)],
                                    ssem)

        scat = {}
        gath = {0: gather(0, 0)}
        for c in range(n_chunks):
            slot = c & 1
            gath[c].wait()
            scat[c] = writeback(c, slot)
            if c + 1 < n_chunks:
                if c - 1 >= 0:
                    scat.pop(c - 1).wait()
                gath[c + 1] = gather(c + 1, slot ^ 1)
        for c in sorted(scat):
            scat[c].wait()

    return k(table, idx3)


def kernel(tokens, charges, aa_table, charge_table):
    B, S = tokens.shape
    V, D = aa_table.shape
    MC = charge_table.shape[0]
    P = S + 1
    n_rows = B * P

    pe = _positional_encoding(P, D)
    aa_z = aa_table.at[0].set(0.0)
    fused = jnp.concatenate(
        [charge_table + pe[0:1],
         (aa_z[None, :, :] + pe[1:, None, :]).reshape(S * V, D)],
        axis=0)


    pos_off = MC + jnp.arange(S, dtype=jnp.int32) * V
    idx = jnp.concatenate(
        [charges[:, None].astype(jnp.int32),
         tokens.astype(jnp.int32) + pos_off[None, :]],
        axis=1)
    idx3 = idx.reshape(NW, -1, CHUNK)

    out = _sc_gather(fused, idx3, n_rows=n_rows, d=D)
    return out.reshape(B, P, D)


# speedup vs baseline: 2.2673x; 2.2673x over previous
import functools

import jax
import jax.numpy as jnp
from jax import lax
from jax.experimental import pallas as pl
from jax.experimental.pallas import tpu as pltpu
from jax.experimental.pallas import tpu_sc as plsc

NC = 2
NS = 16
NW = NC * NS
CHUNK = 96


def _positional_encoding(seq_len, d_model):
    pos = jnp.arange(seq_len, dtype=jnp.float32)[:, None]
    i = jnp.arange(d_model // 2, dtype=jnp.float32)[None, :]
    angle = pos / jnp.power(10000.0, (2.0 * i) / d_model)
    pe = jnp.zeros((seq_len, d_model), dtype=jnp.float32)
    pe = pe.at[:, 0::2].set(jnp.sin(angle))
    pe = pe.at[:, 1::2].set(jnp.cos(angle))
    return pe


@functools.partial(jax.jit, static_argnames=("n_rows", "d"))
def _sc_gather(table, idx3, *, n_rows, d):
    rpw = n_rows // NW
    n_chunks = rpw // CHUNK
    mesh = plsc.VectorSubcoreMesh(core_axis_name="c", subcore_axis_name="s")

    @functools.partial(
        pl.kernel,
        out_type=jax.ShapeDtypeStruct((n_rows, d), jnp.float32),
        mesh=mesh,
        scratch_types=[
            pltpu.VMEM((n_chunks, CHUNK), jnp.int32),
            pltpu.VMEM((2, CHUNK, d), jnp.float32),
            pltpu.SemaphoreType.DMA,
            pltpu.SemaphoreType.DMA,
        ],
    )
    def k(table_hbm, idx_hbm, out_hbm, idx_v, buf_v, gsem, ssem):
        wid = lax.axis_index("s") * NC + lax.axis_index("c")
        base = wid * rpw
        pltpu.sync_copy(idx_hbm.at[wid], idx_v)

        def gather(c, slot):
            return pltpu.async_copy(table_hbm.at[idx_v.at[c]],
                                    buf_v.at[slot], gsem)

        def writeback(c, slot):
            return pltpu.async_copy(buf_v.at[slot],
                                    out_hbm.at[pl.ds(base + c * CHUNK, CHUNK)],
                                    ssem)

        scat = {}
        gath = {0: gather(0, 0)}
        for c in range(n_chunks):
            slot = c & 1
            gath[c].wait()
            scat[c] = writeback(c, slot)
            if c + 1 < n_chunks:
                if c - 1 >= 0:
                    scat.pop(c - 1).wait()
                gath[c + 1] = gather(c + 1, slot ^ 1)
        for c in sorted(scat):
            scat[c].wait()

    return k(table, idx3)


def kernel(tokens, charges, aa_table, charge_table):
    B, S = tokens.shape
    V, D = aa_table.shape
    MC = charge_table.shape[0]
    P = S + 1
    n_rows = B * P

    pe = _positional_encoding(P, D)
    aa_z = aa_table.at[0].set(0.0)
    fused = jnp.concatenate(
        [charge_table + pe[0:1],
         (aa_z[None, :, :] + pe[1:, None, :]).reshape(S * V, D)],
        axis=0)


    pos_off = MC + jnp.arange(S, dtype=jnp.int32) * V
    idx_t = jnp.concatenate(
        [charges[None, :].astype(jnp.int32),
         tokens.astype(jnp.int32).T + pos_off[:, None]],
        axis=0)
    idx3 = idx_t.reshape(NW, -1, CHUNK)

    out = _sc_gather(fused, idx3, n_rows=n_rows, d=D)
    return out.reshape(P, B, D).transpose(1, 0, 2)
